# Initial kernel scaffold; baseline (speedup 1.0000x reference)
import jax
import jax.numpy as jnp
from jax.experimental import pallas as pl


def kernel(x, adj, W1, b1, W2, b2):
    raise NotImplementedError("write your pallas kernel here")


# speedup vs baseline: 9.4675x; 9.4675x over previous
import functools

import jax
import jax.numpy as jnp
from jax import lax
from jax.experimental import pallas as pl
from jax.experimental.pallas import tpu as pltpu
from jax.experimental.pallas import tpu_sc as plsc

N = 10000
F = 256
FH = 128
E = 160000
NC, NS = 2, 16
CHUNK = 128
CPT = 80
EPAD = NS * CPT * CHUNK
NACC = 10240
DUMMY = 10100
ROWS_PT = 640
ROWS_LAST = N - 15 * ROWS_PT

_mesh = plsc.VectorSubcoreMesh(
    core_axis_name="c", subcore_axis_name="s", num_cores=NC, num_subcores=NS)


def _deg_body(dst3, out, deg_sp, dstv, ones_v, zb):
    cid = lax.axis_index("c")
    sid = lax.axis_index("s")
    zrows = NACC // NS
    def zloop(i, _):
        zb[pl.ds(i * 16, 16)] = jnp.zeros((16,), jnp.float32)
        return 0
    lax.fori_loop(0, zrows // 16, zloop, 0)
    pltpu.sync_copy(zb, deg_sp.at[pl.ds(sid * zrows, zrows)])
    def oloop(i, _):
        ones_v[pl.ds(i * 16, 16)] = jnp.ones((16,), jnp.float32)
        return 0
    lax.fori_loop(0, CHUNK // 16, oloop, 0)
    pltpu.sync_copy(dst3.at[sid].at[pl.ds(cid * (CPT // 2), CPT // 2)], dstv)
    plsc.subcore_barrier()
    def dloop(j, _):
        pltpu.sync_copy(ones_v, deg_sp.at[dstv.at[j]], add=True)
        return 0
    lax.fori_loop(0, CPT // 2, dloop, 0)
    plsc.subcore_barrier()
    @pl.when(sid == 0)
    def _():
        pltpu.sync_copy(deg_sp, out.at[cid])


_deg_call = pl.kernel(
    _deg_body,
    out_type=jax.ShapeDtypeStruct((NC, NACC), jnp.float32),
    mesh=_mesh,
    scratch_types=[
        pltpu.VMEM_SHARED((NACC,), jnp.float32),
        pltpu.VMEM((CPT // 2, CHUNK), jnp.int32),
        pltpu.VMEM((CHUNK,), jnp.float32),
        pltpu.VMEM((NACC // NS,), jnp.float32),
    ],
)


def _agg_body(xs, src3, dst3, out, acc, srcv, dstv, rows):
    cid = lax.axis_index("c")
    sid = lax.axis_index("s")
    @pl.when(sid < NS - 1)
    def _():
        pltpu.sync_copy(xs.at[cid].at[pl.ds(sid * ROWS_PT, ROWS_PT)],
                        acc.at[pl.ds(sid * ROWS_PT, ROWS_PT)])
    @pl.when(sid == NS - 1)
    def _():
        pltpu.sync_copy(xs.at[cid].at[pl.ds(sid * ROWS_PT, ROWS_LAST)],
                        acc.at[pl.ds(sid * ROWS_PT, ROWS_LAST)])
    pltpu.sync_copy(src3.at[sid], srcv)
    pltpu.sync_copy(dst3.at[sid], dstv)
    plsc.subcore_barrier()
    def eloop(j, _):
        pltpu.sync_copy(xs.at[cid].at[srcv.at[j]], rows)
        pltpu.sync_copy(rows, acc.at[dstv.at[j]], add=True)
        return 0
    lax.fori_loop(0, CPT, eloop, 0)
    plsc.subcore_barrier()
    @pl.when(sid < NS - 1)
    def _():
        pltpu.sync_copy(acc.at[pl.ds(sid * ROWS_PT, ROWS_PT)],
                        out.at[cid].at[pl.ds(sid * ROWS_PT, ROWS_PT)])
    @pl.when(sid == NS - 1)
    def _():
        pltpu.sync_copy(acc.at[pl.ds(sid * ROWS_PT, ROWS_LAST)],
                        out.at[cid].at[pl.ds(sid * ROWS_PT, ROWS_LAST)])


_agg_call = pl.kernel(
    _agg_body,
    out_type=jax.ShapeDtypeStruct((NC, N, FH), jnp.float32),
    mesh=_mesh,
    scratch_types=[
        pltpu.VMEM_SHARED((NACC, FH), jnp.float32),
        pltpu.VMEM((CPT, CHUNK), jnp.int32),
        pltpu.VMEM((CPT, CHUNK), jnp.int32),
        pltpu.VMEM((CHUNK, FH), jnp.float32),
    ],
)


_RB = 1000


def _scale_body(x_ref, d0_ref, d1_ref, xs_ref, dinv_ref):
    dinv = lax.rsqrt(d0_ref[...] + d1_ref[...] + 1.0)
    dinv_ref[...] = dinv
    xs_ref[...] = (x_ref[...] * dinv)[None]


def _scale_call(x, d0, d1):
    return pl.pallas_call(
        _scale_body,
        grid=(N // _RB, NC),
        in_specs=[
            pl.BlockSpec((_RB, FH), lambda r, h: (r, h)),
            pl.BlockSpec((_RB, 1), lambda r, h: (r, 0)),
            pl.BlockSpec((_RB, 1), lambda r, h: (r, 0)),
        ],
        out_specs=[
            pl.BlockSpec((1, _RB, FH), lambda r, h: (h, r, 0)),
            pl.BlockSpec((_RB, 1), lambda r, h: (r, 0)),
        ],
        out_shape=[
            jax.ShapeDtypeStruct((NC, N, FH), jnp.float32),
            jax.ShapeDtypeStruct((N, 1), jnp.float32),
        ],
    )(x, d0, d1)


def _dense_body(acc_ref, dinv_ref, w1_ref, b1_ref, w2_ref, gs_ref):
    a = jnp.concatenate([acc_ref[0], acc_ref[1]], axis=1)
    dinv = dinv_ref[...]
    t = jnp.dot(a, w1_ref[...], preferred_element_type=jnp.float32)
    h = jnp.maximum(t * dinv + b1_ref[...], 0.0)
    g = jnp.dot(h, w2_ref[...], preferred_element_type=jnp.float32)
    gs = g * dinv
    gs_ref[0] = gs[:, :FH]
    gs_ref[1] = gs[:, FH:]


def _dense_call(acc1, dinv, W1, b1, W2):
    return pl.pallas_call(
        _dense_body,
        grid=(N // _RB,),
        in_specs=[
            pl.BlockSpec((NC, _RB, FH), lambda r: (0, r, 0)),
            pl.BlockSpec((_RB, 1), lambda r: (r, 0)),
            pl.BlockSpec(W1.shape, lambda r: (0, 0)),
            pl.BlockSpec((1, 512), lambda r: (0, 0)),
            pl.BlockSpec(W2.shape, lambda r: (0, 0)),
        ],
        out_specs=pl.BlockSpec((NC, _RB, FH), lambda r: (0, r, 0)),
        out_shape=jax.ShapeDtypeStruct((NC, N, FH), jnp.float32),
    )(acc1, dinv, W1, b1, W2)


def _final_body(acc_ref, dinv_ref, b2_ref, out_ref):
    a = jnp.concatenate([acc_ref[0], acc_ref[1]], axis=1)
    out_ref[...] = a * dinv_ref[...] + b2_ref[...]


def _final_call(acc2, dinv, b2):
    return pl.pallas_call(
        _final_body,
        grid=(N // _RB,),
        in_specs=[
            pl.BlockSpec((NC, _RB, FH), lambda r: (0, r, 0)),
            pl.BlockSpec((_RB, 1), lambda r: (r, 0)),
            pl.BlockSpec((1, F), lambda r: (0, 0)),
        ],
        out_specs=pl.BlockSpec((_RB, F), lambda r: (r, 0)),
        out_shape=jax.ShapeDtypeStruct((N, F), jnp.float32),
    )(acc2, dinv, b2)


def kernel(x, adj, W1, b1, W2, b2):
    src = adj[0].astype(jnp.int32)
    dst = adj[1].astype(jnp.int32)
    npad = EPAD - E
    src3 = jnp.concatenate([src, jnp.zeros((npad,), jnp.int32)]
                           ).reshape(NS, CPT, CHUNK)
    dst3 = jnp.concatenate([dst, jnp.full((npad,), DUMMY, jnp.int32)]
                           ).reshape(NS, CPT, CHUNK)

    degp = _deg_call(dst3)
    d0 = degp[0, :N, None]
    d1 = degp[1, :N, None]
    xs, dinv = _scale_call(x, d0, d1)
    acc1 = _agg_call(xs, src3, dst3)
    gs = _dense_call(acc1, dinv, W1, b1.reshape(1, 512), W2)
    acc2 = _agg_call(gs, src3, dst3)
    return _final_call(acc2, dinv, b2.reshape(1, F))


# speedup vs baseline: 10.7175x; 1.1320x over previous
import functools

import jax
import jax.numpy as jnp
from jax import lax
from jax.experimental import pallas as pl
from jax.experimental.pallas import tpu as pltpu
from jax.experimental.pallas import tpu_sc as plsc

N = 10000
F = 256
FH = 128
E = 160000
NC, NS = 2, 16
CHUNK = 128
CPT = 80
EPAD = NS * CPT * CHUNK
NACC = 10016
NDEG = 10240
DUMMY = 10008
HCPT = CPT // 2
ROWS_PT = 640
ROWS_LAST = N - 15 * ROWS_PT

_mesh = plsc.VectorSubcoreMesh(
    core_axis_name="c", subcore_axis_name="s", num_cores=NC, num_subcores=NS)


def _deg_body(dst3, out, deg_sp, dstv, ones_v, zb):
    cid = lax.axis_index("c")
    sid = lax.axis_index("s")
    zrows = NDEG // NS
    def zloop(i, _):
        zb[pl.ds(i * 16, 16)] = jnp.zeros((16,), jnp.float32)
        return 0
    lax.fori_loop(0, zrows // 16, zloop, 0)
    pltpu.sync_copy(zb, deg_sp.at[pl.ds(sid * zrows, zrows)])
    def oloop(i, _):
        ones_v[pl.ds(i * 16, 16)] = jnp.ones((16,), jnp.float32)
        return 0
    lax.fori_loop(0, CHUNK // 16, oloop, 0)
    pltpu.sync_copy(dst3.at[sid].at[pl.ds(cid * (CPT // 2), CPT // 2)], dstv)
    plsc.subcore_barrier()
    def dloop(j, _):
        pltpu.sync_copy(ones_v, deg_sp.at[dstv.at[j]], add=True)
        return 0
    lax.fori_loop(0, CPT // 2, dloop, 0)
    plsc.subcore_barrier()
    @pl.when(sid == 0)
    def _():
        pltpu.sync_copy(deg_sp, out.at[cid])


_deg_call = pl.kernel(
    _deg_body,
    out_type=jax.ShapeDtypeStruct((NC, NDEG), jnp.float32),
    mesh=_mesh,
    scratch_types=[
        pltpu.VMEM_SHARED((NDEG,), jnp.float32),
        pltpu.VMEM((CPT // 2, CHUNK), jnp.int32),
        pltpu.VMEM((CHUNK,), jnp.float32),
        pltpu.VMEM((NDEG // NS,), jnp.float32),
    ],
)


def _agg_body(xs, src3, dst3, out, acc, srcv, dstv,
              b0, b1, g0, g1, s0, s1):
    bufs = (b0, b1)
    gsem = (g0, g1)
    ssem = (s0, s1)
    cid = lax.axis_index("c")
    sid = lax.axis_index("s")
    @pl.when(sid < NS - 1)
    def _():
        pltpu.sync_copy(xs.at[cid].at[pl.ds(sid * ROWS_PT, ROWS_PT)],
                        acc.at[pl.ds(sid * ROWS_PT, ROWS_PT)])
    @pl.when(sid == NS - 1)
    def _():
        pltpu.sync_copy(xs.at[cid].at[pl.ds(sid * ROWS_PT, ROWS_LAST)],
                        acc.at[pl.ds(sid * ROWS_PT, ROWS_LAST)])
    plsc.subcore_barrier()

    xsh = xs.at[cid]

    def g_start(j, k):
        pltpu.async_copy(xsh.at[srcv.at[j]], bufs[k], gsem[k])

    def g_wait(j, k):
        pltpu.make_async_copy(xsh.at[srcv.at[j]], bufs[k], gsem[k]).wait()

    def s_start(j, k):
        pltpu.async_copy(bufs[k], acc.at[dstv.at[j]], ssem[k], add=True)

    def s_wait(j, k):
        pltpu.make_async_copy(bufs[k], acc.at[dstv.at[j]], ssem[k]).wait()

    for h in range(2):
        pltpu.sync_copy(src3.at[sid].at[pl.ds(h * HCPT, HCPT)], srcv)
        pltpu.sync_copy(dst3.at[sid].at[pl.ds(h * HCPT, HCPT)], dstv)
        g_start(0, 0)
        g_start(1, 1)

        def main(i, _):
            for k in range(2):
                m = 2 * i + k
                g_wait(m, k)
                s_start(m, k)
                s_wait(m, k)
                g_start(m + 2, k)
            return 0
        lax.fori_loop(0, HCPT // 2 - 1, main, 0)

        for k in range(2):
            m = HCPT - 2 + k
            g_wait(m, k)
            s_start(m, k)
            s_wait(m, k)
    plsc.subcore_barrier()
    @pl.when(sid < NS - 1)
    def _():
        pltpu.sync_copy(acc.at[pl.ds(sid * ROWS_PT, ROWS_PT)],
                        out.at[cid].at[pl.ds(sid * ROWS_PT, ROWS_PT)])
    @pl.when(sid == NS - 1)
    def _():
        pltpu.sync_copy(acc.at[pl.ds(sid * ROWS_PT, ROWS_LAST)],
                        out.at[cid].at[pl.ds(sid * ROWS_PT, ROWS_LAST)])


_agg_call = pl.kernel(
    _agg_body,
    out_type=jax.ShapeDtypeStruct((NC, N, FH), jnp.float32),
    mesh=_mesh,
    scratch_types=(
        [
            pltpu.VMEM_SHARED((NACC, FH), jnp.float32),
            pltpu.VMEM((HCPT, CHUNK), jnp.int32),
            pltpu.VMEM((HCPT, CHUNK), jnp.int32),
        ]
        + [pltpu.VMEM((CHUNK, FH), jnp.float32)] * 2
        + [pltpu.SemaphoreType.DMA] * 4
    ),
)


_RB = 1000


def _scale_body(x_ref, d0_ref, d1_ref, xs_ref, dinv_ref):
    dinv = lax.rsqrt(d0_ref[...] + d1_ref[...] + 1.0)
    dinv_ref[...] = dinv
    xs_ref[...] = (x_ref[...] * dinv)[None]


def _scale_call(x, d0, d1):
    return pl.pallas_call(
        _scale_body,
        grid=(N // _RB, NC),
        in_specs=[
            pl.BlockSpec((_RB, FH), lambda r, h: (r, h)),
            pl.BlockSpec((_RB, 1), lambda r, h: (r, 0)),
            pl.BlockSpec((_RB, 1), lambda r, h: (r, 0)),
        ],
        out_specs=[
            pl.BlockSpec((1, _RB, FH), lambda r, h: (h, r, 0)),
            pl.BlockSpec((_RB, 1), lambda r, h: (r, 0)),
        ],
        out_shape=[
            jax.ShapeDtypeStruct((NC, N, FH), jnp.float32),
            jax.ShapeDtypeStruct((N, 1), jnp.float32),
        ],
    )(x, d0, d1)


def _dense_body(acc_ref, dinv_ref, w1_ref, b1_ref, w2_ref, gs_ref):
    a = jnp.concatenate([acc_ref[0], acc_ref[1]], axis=1)
    dinv = dinv_ref[...]
    t = jnp.dot(a, w1_ref[...], preferred_element_type=jnp.float32)
    h = jnp.maximum(t * dinv + b1_ref[...], 0.0)
    g = jnp.dot(h, w2_ref[...], preferred_element_type=jnp.float32)
    gs = g * dinv
    gs_ref[0] = gs[:, :FH]
    gs_ref[1] = gs[:, FH:]


def _dense_call(acc1, dinv, W1, b1, W2):
    return pl.pallas_call(
        _dense_body,
        grid=(N // _RB,),
        in_specs=[
            pl.BlockSpec((NC, _RB, FH), lambda r: (0, r, 0)),
            pl.BlockSpec((_RB, 1), lambda r: (r, 0)),
            pl.BlockSpec(W1.shape, lambda r: (0, 0)),
            pl.BlockSpec((1, 512), lambda r: (0, 0)),
            pl.BlockSpec(W2.shape, lambda r: (0, 0)),
        ],
        out_specs=pl.BlockSpec((NC, _RB, FH), lambda r: (0, r, 0)),
        out_shape=jax.ShapeDtypeStruct((NC, N, FH), jnp.float32),
    )(acc1, dinv, W1, b1, W2)


def _final_body(acc_ref, dinv_ref, b2_ref, out_ref):
    a = jnp.concatenate([acc_ref[0], acc_ref[1]], axis=1)
    out_ref[...] = a * dinv_ref[...] + b2_ref[...]


def _final_call(acc2, dinv, b2):
    return pl.pallas_call(
        _final_body,
        grid=(N // _RB,),
        in_specs=[
            pl.BlockSpec((NC, _RB, FH), lambda r: (0, r, 0)),
            pl.BlockSpec((_RB, 1), lambda r: (r, 0)),
            pl.BlockSpec((1, F), lambda r: (0, 0)),
        ],
        out_specs=pl.BlockSpec((_RB, F), lambda r: (r, 0)),
        out_shape=jax.ShapeDtypeStruct((N, F), jnp.float32),
    )(acc2, dinv, b2)


def kernel(x, adj, W1, b1, W2, b2):
    src = adj[0].astype(jnp.int32)
    dst = adj[1].astype(jnp.int32)
    npad = EPAD - E
    src3 = jnp.concatenate([src, jnp.zeros((npad,), jnp.int32)]
                           ).reshape(NS, CPT, CHUNK)
    dst3 = jnp.concatenate([dst, jnp.full((npad,), DUMMY, jnp.int32)]
                           ).reshape(NS, CPT, CHUNK)

    degp = _deg_call(dst3)
    d0 = degp[0, :N, None]
    d1 = degp[1, :N, None]
    xs, dinv = _scale_call(x, d0, d1)
    acc1 = _agg_call(xs, src3, dst3)
    gs = _dense_call(acc1, dinv, W1, b1.reshape(1, 512), W2)
    acc2 = _agg_call(gs, src3, dst3)
    return _final_call(acc2, dinv, b2.reshape(1, F))
